# baseline (speedup 1.0000x reference)
import jax
import jax.numpy as jnp
from jax.experimental import pallas as pl
from jax.experimental.pallas import tpu as pltpu

_E = 3200000
_RI = _E // 64
_RV = _E // 128
_G = 5
_BI = _RI // _G


def _iview(a):
    return a.reshape(_RI // 2, 128, 2).swapaxes(1, 2).reshape(_RI, 128)


def _body(m_ref, a2i, a1i_hbm, a1v_hbm, a2v_hbm, oi_hbm, ov_hbm,
          buf, sem_big, sem_out):
    i = pl.program_id(0)

    @pl.when(i == 0)
    def _start_big_copies():
        pltpu.make_async_copy(a1i_hbm, oi_hbm.at[0], sem_big.at[0]).start()
        pltpu.make_async_copy(a1v_hbm, ov_hbm.at[0], sem_big.at[1]).start()
        pltpu.make_async_copy(a2v_hbm, ov_hbm.at[1], sem_big.at[2]).start()

    slot = jax.lax.rem(i, 2)

    @pl.when(i >= 2)
    def _wait_prev():
        pltpu.make_async_copy(
            buf.at[slot], oi_hbm.at[1].at[pl.ds((i - 2) * _BI, _BI)],
            sem_out.at[slot]).wait()

    buf[slot] = a2i[...] + m_ref[0]
    copy = pltpu.make_async_copy(
        buf.at[slot], oi_hbm.at[1].at[pl.ds(i * _BI, _BI)], sem_out.at[slot])
    copy.start()

    @pl.when(i == _G - 1)
    def _drain():
        pltpu.make_async_copy(
            buf.at[1 - slot], oi_hbm.at[1].at[pl.ds((i - 1) * _BI, _BI)],
            sem_out.at[1 - slot]).wait()
        copy.wait()
        pltpu.make_async_copy(a1i_hbm, oi_hbm.at[0], sem_big.at[0]).wait()
        pltpu.make_async_copy(a1v_hbm, ov_hbm.at[0], sem_big.at[1]).wait()
        pltpu.make_async_copy(a2v_hbm, ov_hbm.at[1], sem_big.at[2]).wait()


def kernel(a1_indices, a1_values, a2_indices, a2_values, M):
    idt = a1_indices.dtype
    a1i = _iview(a1_indices)
    a2i = _iview(a2_indices)
    a1v = a1_values.reshape(_RV, 128)
    a2v = a2_values.reshape(_RV, 128)
    m = jnp.asarray(M, idt).reshape(1)

    oi, ov = pl.pallas_call(
        _body,
        grid=(_G,),
        in_specs=[
            pl.BlockSpec(memory_space=pltpu.SMEM),
            pl.BlockSpec((_BI, 128), lambda i: (i, 0)),
            pl.BlockSpec(memory_space=pltpu.MemorySpace.HBM),
            pl.BlockSpec(memory_space=pltpu.MemorySpace.HBM),
            pl.BlockSpec(memory_space=pltpu.MemorySpace.HBM),
        ],
        out_specs=[
            pl.BlockSpec(memory_space=pltpu.MemorySpace.HBM),
            pl.BlockSpec(memory_space=pltpu.MemorySpace.HBM),
        ],
        out_shape=[
            jax.ShapeDtypeStruct((2, _RI, 128), idt),
            jax.ShapeDtypeStruct((2, _RV, 128), a1_values.dtype),
        ],
        scratch_shapes=[
            pltpu.VMEM((2, _BI, 128), idt),
            pltpu.SemaphoreType.DMA((3,)),
            pltpu.SemaphoreType.DMA((2,)),
        ],
    )(m, a2i, a1i, a1v, a2v)

    new_inds = (oi.reshape(2 * _RI // 2, 2, 128)
                  .swapaxes(1, 2)
                  .reshape(2 * _E, 2))
    new_vals = ov.reshape(2 * _E)
    return new_inds, new_vals


# speedup vs baseline: 31.3510x; 31.3510x over previous
import jax
import jax.numpy as jnp
from jax.experimental import pallas as pl
from jax.experimental.pallas import tpu as pltpu

_E = 3200000
_RI = _E // 64
_RV = _E // 128
_G = 5
_BI = _RI // _G
_BV = _RV // _G


def _iview(a):
    return a.reshape(_RI // 2, 128, 2).swapaxes(1, 2).reshape(_RI, 128)


def _body(m_ref, a1i, a2i, a1v, a2v, oi, ov):
    oi[0] = a1i[...]
    oi[1] = a2i[...] + m_ref[0]
    ov[0] = a1v[...]
    ov[1] = a2v[...]


def kernel(a1_indices, a1_values, a2_indices, a2_values, M):
    idt = a1_indices.dtype
    a1i = _iview(a1_indices)
    a2i = _iview(a2_indices)
    a1v = a1_values.reshape(_RV, 128)
    a2v = a2_values.reshape(_RV, 128)
    m = jnp.asarray(M, idt).reshape(1)

    oi, ov = pl.pallas_call(
        _body,
        grid=(_G,),
        in_specs=[
            pl.BlockSpec(memory_space=pltpu.SMEM),
            pl.BlockSpec((_BI, 128), lambda i: (i, 0)),
            pl.BlockSpec((_BI, 128), lambda i: (i, 0)),
            pl.BlockSpec((_BV, 128), lambda i: (i, 0)),
            pl.BlockSpec((_BV, 128), lambda i: (i, 0)),
        ],
        out_specs=[
            pl.BlockSpec((2, _BI, 128), lambda i: (0, i, 0)),
            pl.BlockSpec((2, _BV, 128), lambda i: (0, i, 0)),
        ],
        out_shape=[
            jax.ShapeDtypeStruct((2, _RI, 128), idt),
            jax.ShapeDtypeStruct((2, _RV, 128), a1_values.dtype),
        ],
    )(m, a1i, a2i, a1v, a2v)

    new_inds = (oi.reshape(2 * _RI // 2, 2, 128)
                  .swapaxes(1, 2)
                  .reshape(2 * _E, 2))
    new_vals = ov.reshape(2 * _E)
    return new_inds, new_vals
